# Initial kernel scaffold; baseline (speedup 1.0000x reference)
import jax
import jax.numpy as jnp
from jax.experimental import pallas as pl


def kernel(x, proj_w, proj_b, a_in, w, kw1, kb1, kw2, kb2, kw3, kb3, bw1, bb1, bw2, bb2, bw3, bb3, a_out):
    raise NotImplementedError("write your pallas kernel here")


# speedup vs baseline: 6.0183x; 6.0183x over previous
import jax
import jax.numpy as jnp
from jax import lax
from jax.experimental import pallas as pl
from jax.experimental.pallas import tpu as pltpu

C = 16
D = 64
HW = 64 * 64
N = D * HW
PAD = 2 * HW
NP = N + 2 * PAD
K3N = 27
EPS = 1e-5
KM_ITERS = 15
NUM_K = 4
TAPS = [(kd, kh, kw) for kd in (-1, 0, 1) for kh in (-1, 0, 1)
        for kw in (-1, 0, 1)]

_CP = getattr(pltpu, "CompilerParams", None)
if _CP is None:
    _CP = pltpu.TPUCompilerParams


def _hw_masks():
    lane = lax.broadcasted_iota(jnp.int32, (1, HW), 1)
    h = lane // 64
    w = lane - h * 64
    masks = {}
    for kh in (-1, 0, 1):
        for kw in (-1, 0, 1):
            conds = []
            if kh == -1:
                conds.append(h >= 1)
            if kh == 1:
                conds.append(h <= 62)
            if kw == -1:
                conds.append(w >= 1)
            if kw == 1:
                conds.append(w <= 62)
            if not conds:
                masks[(kh, kw)] = None
            else:
                m = conds[0]
                for cnd in conds[1:]:
                    m = jnp.logical_and(m, cnd)
                masks[(kh, kw)] = jnp.where(m, 1.0, 0.0).astype(jnp.float32)
    return masks


def _conv_stats_kernel(xp_ref, w1_ref, b_ref, y_ref, st_ref):
    i = pl.program_id(0)
    base = PAD + i * HW
    masks = _hw_masks()
    win = xp_ref[:, pl.ds(base - 4224, HW + 8448)]
    parts = []
    for (kd, kh, kw) in TAPS:
        off = 4224 + kd * HW + kh * 64 + kw
        sl = win[:, off:off + HW]
        mf = masks[(kh, kw)]
        if mf is not None:
            sl = sl * mf
        parts.append(sl)
    p = jnp.concatenate(parts, axis=0)
    y = jnp.dot(w1_ref[...], p,
                preferred_element_type=jnp.float32) + b_ref[...]
    y_ref[0] = y
    st_ref[0, :, 0:1] = jnp.sum(y, axis=1, keepdims=True)
    st_ref[0, :, 1:2] = jnp.sum(y * y, axis=1, keepdims=True)


def _norm_box_kernel(yp_ref, mu_ref, inv_ref, a_ref, yn_ref, xm_ref):
    i = pl.program_id(0)
    base = PAD + i * HW
    mu = mu_ref[...]
    inv = inv_ref[...]
    a = a_ref[0, 0]
    masks = _hw_masks()
    acc = jnp.zeros((C, HW), jnp.float32)
    win = yp_ref[:, pl.ds(base - 4224, HW + 8448)]
    for (kd, kh, kw) in TAPS:
        off = 4224 + kd * HW + kh * 64 + kw
        sl = win[:, off:off + HW]
        v = (sl - mu) * inv
        v = jnp.where(v >= 0, v, a * v)
        if kd == 0 and kh == 0 and kw == 0:
            yn_ref[0] = v
        mf = masks[(kh, kw)]
        if kd != 0:
            dn = i + kd
            fd = jnp.where(jnp.logical_and(dn >= 0, dn < D), 1.0,
                           0.0).astype(jnp.float32)
            mf = fd if mf is None else mf * fd
        if mf is not None:
            v = v * mf
        acc = acc + v
    xm_ref[0] = acc * (1.0 / 27.0)


def _kmeans_mlp_kernel(pts_ref, kw1_ref, kb1_ref, kw2_ref, kb2_ref,
                       kw3_ref, kb3_ref, bw1_ref, bb1_ref, bw2_ref,
                       bb2_ref, bw3_ref, bb3_ref,
                       asg_ref, modt_ref, bvt_ref):
    pn = pts_ref[:, 0, :] * pts_ref[:, 0, :]
    for c in range(1, C):
        pn = pn + pts_ref[:, c, :] * pts_ref[:, c, :]

    cent0 = jnp.concatenate(
        [pts_ref[0, :, 0:1], pts_ref[21, :, 1365:1366],
         pts_ref[42, :, 2730:2731], pts_ref[63, :, 4095:4096]],
        axis=1)

    CSZ = 8

    def _scalars(cent):
        cs = [[cent[c, j] for j in range(NUM_K)] for c in range(C)]
        cn = [sum(cs[c][j] * cs[c][j] for c in range(C))
              for j in range(NUM_K)]
        return cs, cn

    def chunk_assign(pch, pnch, cs, cn):
        best = None
        bi = None
        for j in range(NUM_K):
            dot = pch[0] * cs[0][j]
            for c in range(1, C):
                dot = dot + pch[c] * cs[c][j]
            dj = pnch - 2.0 * dot + cn[j]
            if j == 0:
                best = dj
                bi = jnp.zeros(dj.shape, jnp.int32)
            else:
                flip = dj < best
                bi = jnp.where(flip, j, bi)
                best = jnp.where(flip, dj, best)
        return bi

    def body(_, cent):
        cs, cn = _scalars(cent)
        s_acc = None
        cnt_acc = None
        for d0 in range(0, D, CSZ):
            pch = [pts_ref[d0:d0 + CSZ, c, :] for c in range(C)]
            bi = chunk_assign(pch, pn[d0:d0 + CSZ, :], cs, cn)
            cols = []
            cnts = []
            for j in range(NUM_K):
                mf = jnp.where(bi == j, 1.0, 0.0).astype(jnp.float32)
                cnts.append(jnp.sum(mf, axis=(0, 1), keepdims=True))
                col = [jnp.sum(pch[c] * mf, axis=(0, 1), keepdims=True)
                       for c in range(C)]
                cols.append(jnp.concatenate(col, axis=0))
            s = jnp.concatenate(cols, axis=1)
            cnt = jnp.concatenate(cnts, axis=1)
            s_acc = s if s_acc is None else s_acc + s
            cnt_acc = cnt if cnt_acc is None else cnt_acc + cnt
        return jnp.where(cnt_acc > 0, s_acc / jnp.maximum(cnt_acc, 1.0),
                         cent)

    cent = lax.fori_loop(0, KM_ITERS, body, cent0)
    cs, cn = _scalars(cent)
    for d0 in range(0, D, CSZ):
        pch = [pts_ref[d0:d0 + CSZ, c, :] for c in range(C)]
        asg_ref[d0:d0 + CSZ, :] = chunk_assign(pch, pn[d0:d0 + CSZ, :],
                                               cs, cn)

    def dgt(a_ref_, b):
        return lax.dot_general(a_ref_[...], b, (((1,), (0,)), ((), ())),
                               preferred_element_type=jnp.float32)

    h1 = jnp.maximum(dgt(kw1_ref, cent) + kb1_ref[...], 0.0)
    h2 = jnp.maximum(dgt(kw2_ref, h1) + kb2_ref[...], 0.0)
    modt_ref[...] = jax.nn.sigmoid(dgt(kw3_ref, h2) + kb3_ref[...])
    g1 = jnp.maximum(dgt(bw1_ref, cent) + bb1_ref[...], 0.0)
    g2 = jnp.maximum(dgt(bw2_ref, g1) + bb2_ref[...], 0.0)
    bvt_ref[...] = dgt(bw3_ref, g2) + bb3_ref[...]


def _dynconv_kernel(ynp_ref, asg_ref, modt_ref, bvt_ref, w2_ref, x_ref,
                    a_ref, o_ref):
    i = pl.program_id(0)
    base = PAD + i * HW
    a2 = asg_ref[0, :, :]
    masks = _hw_masks()
    modn = None
    bn = None
    for j in range(NUM_K):
        mjf = jnp.where(a2 == j, 1.0, 0.0).astype(jnp.float32)
        mterm = modt_ref[:, j:j + 1] * mjf
        bterm = bvt_ref[:, j:j + 1] * mjf
        modn = mterm if modn is None else modn + mterm
        bn = bterm if bn is None else bn + bterm
    parts = []
    win = ynp_ref[:, pl.ds(base - 4224, HW + 8448)]
    for t, (kd, kh, kw) in enumerate(TAPS):
        off = 4224 + kd * HW + kh * 64 + kw
        sl = win[:, off:off + HW]
        mf = masks[(kh, kw)]
        if mf is not None:
            sl = sl * mf
        parts.append(sl * modn[t:t + 1, :])
    p = jnp.concatenate(parts, axis=0)
    out = jnp.dot(w2_ref[...], p, preferred_element_type=jnp.float32)
    out = out + bn
    aa = a_ref[0, 0]
    out = jnp.where(out >= 0, out, aa * out)
    o_ref[0] = out + x_ref[0]


def _full(shape):
    return pl.BlockSpec(shape, lambda i: tuple(0 for _ in shape))


def kernel(x, proj_w, proj_b, a_in, w, kw1, kb1, kw2, kb2, kw3, kb3,
           bw1, bb1, bw2, bb2, bw3, bb3, a_out):
    f32 = jnp.float32
    xf = x.reshape(C, N)
    xfp = jnp.pad(xf, ((0, 0), (PAD, PAD)))
    w1 = proj_w.reshape(C, C, K3N).transpose(0, 2, 1).reshape(C, K3N * C)
    pb = proj_b.reshape(C, 1)

    cp = _CP(dimension_semantics=("parallel",),
             vmem_limit_bytes=50 * 1024 * 1024)

    y, st = pl.pallas_call(
        _conv_stats_kernel,
        grid=(D,),
        in_specs=[_full((C, NP)), _full((C, K3N * C)), _full((C, 1))],
        out_specs=[pl.BlockSpec((1, C, HW), lambda i: (i, 0, 0)),
                   pl.BlockSpec((1, C, 2), lambda i: (i, 0, 0))],
        out_shape=[jax.ShapeDtypeStruct((D, C, HW), f32),
                   jax.ShapeDtypeStruct((D, C, 2), f32)],
        compiler_params=cp,
    )(xfp, w1, pb)

    ssum = jnp.sum(st[:, :, 0], axis=0)
    ssq = jnp.sum(st[:, :, 1], axis=0)
    mu = ssum / N
    var = ssq / N - mu * mu
    inv = lax.rsqrt(var + EPS)

    yfp = jnp.pad(y.transpose(1, 0, 2).reshape(C, N), ((0, 0), (PAD, PAD)))
    yn, xm = pl.pallas_call(
        _norm_box_kernel,
        grid=(D,),
        in_specs=[_full((C, NP)), _full((C, 1)), _full((C, 1)),
                  _full((1, 1))],
        out_specs=[pl.BlockSpec((1, C, HW), lambda i: (i, 0, 0)),
                   pl.BlockSpec((1, C, HW), lambda i: (i, 0, 0))],
        out_shape=[jax.ShapeDtypeStruct((D, C, HW), f32),
                   jax.ShapeDtypeStruct((D, C, HW), f32)],
        compiler_params=cp,
    )(yfp, mu.reshape(C, 1), inv.reshape(C, 1),
      a_in.reshape(1, 1).astype(f32))

    asg, modt, bvt = pl.pallas_call(
        _kmeans_mlp_kernel,
        out_shape=[jax.ShapeDtypeStruct((D, HW), jnp.int32),
                   jax.ShapeDtypeStruct((K3N, NUM_K), f32),
                   jax.ShapeDtypeStruct((C, NUM_K), f32)],
        compiler_params=_CP(vmem_limit_bytes=50 * 1024 * 1024),
    )(xm, kw1, kb1.reshape(-1, 1), kw2, kb2.reshape(-1, 1),
      kw3, kb3.reshape(-1, 1), bw1, bb1.reshape(-1, 1),
      bw2, bb2.reshape(-1, 1), bw3, bb3.reshape(-1, 1))

    ynp = jnp.pad(yn.transpose(1, 0, 2).reshape(C, N), ((0, 0), (PAD, PAD)))
    w2 = w.transpose(0, 2, 1).reshape(C, K3N * C)
    out = pl.pallas_call(
        _dynconv_kernel,
        grid=(D,),
        in_specs=[_full((C, NP)),
                  pl.BlockSpec((1, 1, HW), lambda i: (i, 0, 0)),
                  _full((K3N, NUM_K)), _full((C, NUM_K)),
                  _full((C, K3N * C)),
                  pl.BlockSpec((1, C, HW), lambda i: (i, 0, 0)),
                  _full((1, 1))],
        out_specs=pl.BlockSpec((1, C, HW), lambda i: (i, 0, 0)),
        out_shape=jax.ShapeDtypeStruct((D, C, HW), f32),
        compiler_params=cp,
    )(ynp, asg.reshape(D, 1, HW), modt, bvt, w2,
      xf.reshape(C, D, HW).transpose(1, 0, 2),
      a_out.reshape(1, 1).astype(f32))

    return out.transpose(1, 0, 2).reshape(1, C, D, 64, 64)


# speedup vs baseline: 6.4986x; 1.0798x over previous
import jax
import jax.numpy as jnp
from jax import lax
from jax.experimental import pallas as pl
from jax.experimental.pallas import tpu as pltpu

C = 16
D = 64
HW = 64 * 64
N = D * HW
PAD = 2 * HW
NP = N + 2 * PAD
K3N = 27
EPS = 1e-5
KM_ITERS = 15
NUM_K = 4
TAPS = [(kd, kh, kw) for kd in (-1, 0, 1) for kh in (-1, 0, 1)
        for kw in (-1, 0, 1)]

_CP = getattr(pltpu, "CompilerParams", None)
if _CP is None:
    _CP = pltpu.TPUCompilerParams


def _hw_masks():
    lane = lax.broadcasted_iota(jnp.int32, (1, HW), 1)
    h = lane // 64
    w = lane - h * 64
    masks = {}
    for kh in (-1, 0, 1):
        for kw in (-1, 0, 1):
            conds = []
            if kh == -1:
                conds.append(h >= 1)
            if kh == 1:
                conds.append(h <= 62)
            if kw == -1:
                conds.append(w >= 1)
            if kw == 1:
                conds.append(w <= 62)
            if not conds:
                masks[(kh, kw)] = None
            else:
                m = conds[0]
                for cnd in conds[1:]:
                    m = jnp.logical_and(m, cnd)
                masks[(kh, kw)] = jnp.where(m, 1.0, 0.0).astype(jnp.float32)
    return masks


def _conv_stats_kernel(xp_ref, w1_ref, b_ref, y_ref, st_ref):
    j = pl.program_id(0)
    is_pad = jnp.logical_or(j < 2, j >= D + 2)

    @pl.when(is_pad)
    def _():
        y_ref[...] = jnp.zeros((C, HW), jnp.float32)
        st_ref[...] = jnp.zeros((1, C, 2), jnp.float32)

    @pl.when(jnp.logical_not(is_pad))
    def _():
        base = j * HW
        masks = _hw_masks()
        win = xp_ref[:, pl.ds(base - 4224, HW + 8448)]
        parts = []
        for (kd, kh, kw) in TAPS:
            off = 4224 + kd * HW + kh * 64 + kw
            sl = win[:, off:off + HW]
            mf = masks[(kh, kw)]
            if mf is not None:
                sl = sl * mf
            parts.append(sl)
        p = jnp.concatenate(parts, axis=0)
        y = jnp.dot(w1_ref[...], p,
                    preferred_element_type=jnp.float32) + b_ref[...]
        y_ref[...] = y
        st_ref[0, :, 0:1] = jnp.sum(y, axis=1, keepdims=True)
        st_ref[0, :, 1:2] = jnp.sum(y * y, axis=1, keepdims=True)


def _norm_box_kernel(yp_ref, mu_ref, inv_ref, a_ref, yn_ref, xm_ref):
    j = pl.program_id(0)
    is_pad = jnp.logical_or(j < 2, j >= D + 2)

    @pl.when(is_pad)
    def _():
        yn_ref[...] = jnp.zeros((C, HW), jnp.float32)
        xm_ref[...] = jnp.zeros((1, C, HW), jnp.float32)

    @pl.when(jnp.logical_not(is_pad))
    def _():
        base = j * HW
        mu = mu_ref[...]
        inv = inv_ref[...]
        a = a_ref[0, 0]
        masks = _hw_masks()
        acc = jnp.zeros((C, HW), jnp.float32)
        win = yp_ref[:, pl.ds(base - 4224, HW + 8448)]
        for (kd, kh, kw) in TAPS:
            off = 4224 + kd * HW + kh * 64 + kw
            sl = win[:, off:off + HW]
            v = (sl - mu) * inv
            v = jnp.where(v >= 0, v, a * v)
            if kd == 0 and kh == 0 and kw == 0:
                yn_ref[...] = v
            mf = masks[(kh, kw)]
            if kd != 0:
                dn = j - 2 + kd
                fd = jnp.where(jnp.logical_and(dn >= 0, dn < D), 1.0,
                               0.0).astype(jnp.float32)
                mf = fd if mf is None else mf * fd
            if mf is not None:
                v = v * mf
            acc = acc + v
        xm_ref[0] = acc * (1.0 / 27.0)


def _kmeans_mlp_kernel(pts_ref, kw1_ref, kb1_ref, kw2_ref, kb2_ref,
                       kw3_ref, kb3_ref, bw1_ref, bb1_ref, bw2_ref,
                       bb2_ref, bw3_ref, bb3_ref,
                       asg_ref, modt_ref, bvt_ref):
    pn = pts_ref[2:D + 2, 0, :] * pts_ref[2:D + 2, 0, :]
    for c in range(1, C):
        pn = pn + pts_ref[2:D + 2, c, :] * pts_ref[2:D + 2, c, :]

    cent0 = jnp.concatenate(
        [pts_ref[2, :, 0:1], pts_ref[23, :, 1365:1366],
         pts_ref[44, :, 2730:2731], pts_ref[65, :, 4095:4096]],
        axis=1)

    CSZ = 8

    def _scalars(cent):
        cs = [[cent[c, j] for j in range(NUM_K)] for c in range(C)]
        cn = [sum(cs[c][j] * cs[c][j] for c in range(C))
              for j in range(NUM_K)]
        return cs, cn

    def chunk_assign(pch, pnch, cs, cn):
        best = None
        bi = None
        for j in range(NUM_K):
            dot = pch[0] * cs[0][j]
            for c in range(1, C):
                dot = dot + pch[c] * cs[c][j]
            dj = pnch - 2.0 * dot + cn[j]
            if j == 0:
                best = dj
                bi = jnp.zeros(dj.shape, jnp.int32)
            else:
                flip = dj < best
                bi = jnp.where(flip, j, bi)
                best = jnp.where(flip, dj, best)
        return bi

    def body(_, cent):
        cs, cn = _scalars(cent)
        s_acc = None
        cnt_acc = None
        for d0 in range(0, D, CSZ):
            pch = [pts_ref[d0 + 2:d0 + 2 + CSZ, c, :] for c in range(C)]
            bi = chunk_assign(pch, pn[d0:d0 + CSZ, :], cs, cn)
            cols = []
            cnts = []
            for j in range(NUM_K):
                mf = jnp.where(bi == j, 1.0, 0.0).astype(jnp.float32)
                cnts.append(jnp.sum(mf, axis=(0, 1), keepdims=True))
                col = [jnp.sum(pch[c] * mf, axis=(0, 1), keepdims=True)
                       for c in range(C)]
                cols.append(jnp.concatenate(col, axis=0))
            s = jnp.concatenate(cols, axis=1)
            cnt = jnp.concatenate(cnts, axis=1)
            s_acc = s if s_acc is None else s_acc + s
            cnt_acc = cnt if cnt_acc is None else cnt_acc + cnt
        return jnp.where(cnt_acc > 0, s_acc / jnp.maximum(cnt_acc, 1.0),
                         cent)

    cent = lax.fori_loop(0, KM_ITERS, body, cent0)
    cs, cn = _scalars(cent)
    for d0 in range(0, D, CSZ):
        pch = [pts_ref[d0 + 2:d0 + 2 + CSZ, c, :] for c in range(C)]
        asg_ref[d0:d0 + CSZ, :] = chunk_assign(pch, pn[d0:d0 + CSZ, :],
                                               cs, cn)

    def dgt(a_ref_, b):
        return lax.dot_general(a_ref_[...], b, (((1,), (0,)), ((), ())),
                               preferred_element_type=jnp.float32)

    h1 = jnp.maximum(dgt(kw1_ref, cent) + kb1_ref[...], 0.0)
    h2 = jnp.maximum(dgt(kw2_ref, h1) + kb2_ref[...], 0.0)
    modt_ref[...] = jax.nn.sigmoid(dgt(kw3_ref, h2) + kb3_ref[...])
    g1 = jnp.maximum(dgt(bw1_ref, cent) + bb1_ref[...], 0.0)
    g2 = jnp.maximum(dgt(bw2_ref, g1) + bb2_ref[...], 0.0)
    bvt_ref[...] = dgt(bw3_ref, g2) + bb3_ref[...]


def _dynconv_kernel(ynp_ref, asg_ref, modt_ref, bvt_ref, w2_ref, x_ref,
                    a_ref, o_ref):
    i = pl.program_id(0)
    base = PAD + i * HW
    a2 = asg_ref[0, :, :]
    masks = _hw_masks()
    modn = None
    bn = None
    for j in range(NUM_K):
        mjf = jnp.where(a2 == j, 1.0, 0.0).astype(jnp.float32)
        mterm = modt_ref[:, j:j + 1] * mjf
        bterm = bvt_ref[:, j:j + 1] * mjf
        modn = mterm if modn is None else modn + mterm
        bn = bterm if bn is None else bn + bterm
    parts = []
    win = ynp_ref[:, pl.ds(base - 4224, HW + 8448)]
    for t, (kd, kh, kw) in enumerate(TAPS):
        off = 4224 + kd * HW + kh * 64 + kw
        sl = win[:, off:off + HW]
        mf = masks[(kh, kw)]
        if mf is not None:
            sl = sl * mf
        parts.append(sl * modn[t:t + 1, :])
    p = jnp.concatenate(parts, axis=0)
    out = jnp.dot(w2_ref[...], p, preferred_element_type=jnp.float32)
    out = out + bn
    aa = a_ref[0, 0]
    out = jnp.where(out >= 0, out, aa * out)
    o_ref[...] = out + x_ref[...]


def _full(shape):
    return pl.BlockSpec(shape, lambda i: tuple(0 for _ in shape))


def kernel(x, proj_w, proj_b, a_in, w, kw1, kb1, kw2, kb2, kw3, kb3,
           bw1, bb1, bw2, bb2, bw3, bb3, a_out):
    f32 = jnp.float32
    xf = x.reshape(C, N)
    xfp = jnp.pad(xf, ((0, 0), (PAD, PAD)))
    w1 = proj_w.reshape(C, C, K3N).transpose(0, 2, 1).reshape(C, K3N * C)
    pb = proj_b.reshape(C, 1)

    cp = _CP(dimension_semantics=("parallel",),
             vmem_limit_bytes=50 * 1024 * 1024)

    y, st = pl.pallas_call(
        _conv_stats_kernel,
        grid=(D + 4,),
        in_specs=[_full((C, NP)), _full((C, K3N * C)), _full((C, 1))],
        out_specs=[pl.BlockSpec((C, HW), lambda j: (0, j)),
                   pl.BlockSpec((1, C, 2), lambda j: (j, 0, 0))],
        out_shape=[jax.ShapeDtypeStruct((C, NP), f32),
                   jax.ShapeDtypeStruct((D + 4, C, 2), f32)],
        compiler_params=cp,
    )(xfp, w1, pb)

    ssum = jnp.sum(st[:, :, 0], axis=0)
    ssq = jnp.sum(st[:, :, 1], axis=0)
    mu = ssum / N
    var = ssq / N - mu * mu
    inv = lax.rsqrt(var + EPS)

    yn, xm = pl.pallas_call(
        _norm_box_kernel,
        grid=(D + 4,),
        in_specs=[_full((C, NP)), _full((C, 1)), _full((C, 1)),
                  _full((1, 1))],
        out_specs=[pl.BlockSpec((C, HW), lambda j: (0, j)),
                   pl.BlockSpec((1, C, HW), lambda j: (j, 0, 0))],
        out_shape=[jax.ShapeDtypeStruct((C, NP), f32),
                   jax.ShapeDtypeStruct((D + 4, C, HW), f32)],
        compiler_params=cp,
    )(y, mu.reshape(C, 1), inv.reshape(C, 1),
      a_in.reshape(1, 1).astype(f32))

    asg, modt, bvt = pl.pallas_call(
        _kmeans_mlp_kernel,
        out_shape=[jax.ShapeDtypeStruct((D, HW), jnp.int32),
                   jax.ShapeDtypeStruct((K3N, NUM_K), f32),
                   jax.ShapeDtypeStruct((C, NUM_K), f32)],
        compiler_params=_CP(vmem_limit_bytes=50 * 1024 * 1024),
    )(xm, kw1, kb1.reshape(-1, 1), kw2, kb2.reshape(-1, 1),
      kw3, kb3.reshape(-1, 1), bw1, bb1.reshape(-1, 1),
      bw2, bb2.reshape(-1, 1), bw3, bb3.reshape(-1, 1))

    w2 = w.transpose(0, 2, 1).reshape(C, K3N * C)
    out = pl.pallas_call(
        _dynconv_kernel,
        grid=(D,),
        in_specs=[_full((C, NP)),
                  pl.BlockSpec((1, 1, HW), lambda i: (i, 0, 0)),
                  _full((K3N, NUM_K)), _full((C, NUM_K)),
                  _full((C, K3N * C)),
                  pl.BlockSpec((C, HW), lambda i: (0, i)),
                  _full((1, 1))],
        out_specs=pl.BlockSpec((C, HW), lambda i: (0, i)),
        out_shape=jax.ShapeDtypeStruct((C, N), f32),
        compiler_params=cp,
    )(yn, asg.reshape(D, 1, HW), modt, bvt, w2, xf,
      a_out.reshape(1, 1).astype(f32))

    return out.reshape(1, C, D, 64, 64)
